# baseline (speedup 1.0000x reference)
import functools

import jax
import jax.numpy as jnp
from jax import lax
from jax.experimental import pallas as pl
from jax.experimental.pallas import tpu as pltpu
from jax.experimental.pallas import tpu_sc as plsc

N = 100000
F = 200000
D = 3
ITERS = 5

NC = 2
NS = 16
F_PAD = 204800
FH = F_PAD // 2
N_PAD = 102400
NR = 200
BF = 2048
CHE = FH // (NC * NS)
CHN = N_PAD // NS


def _rowi():
    return lax.broadcasted_iota(jnp.int32, (8, 1), 0)


def _mk_mask67():
    return (_rowi() < 6).astype(jnp.float32)


def _bc(row):
    return jnp.broadcast_to(row, (8, row.shape[1]))


def _pair_normalize(v):
    even = (_rowi() % 2) == 0
    sw = jnp.where(even, jnp.roll(v, -1, axis=0), jnp.roll(v, 1, axis=0))
    m = jnp.maximum(v, sw)
    l = m + jnp.log1p(jnp.exp(-jnp.abs(v - sw)))
    return v - l


def _expand_msgs(v2f):
    r = _rowi()
    v0 = jnp.where((r >> 2) & 1 == 0, _bc(v2f[0:1]), _bc(v2f[1:2]))
    v1 = jnp.where((r >> 1) & 1 == 0, _bc(v2f[2:3]), _bc(v2f[3:4]))
    v2 = jnp.where(r & 1 == 0, _bc(v2f[4:5]), _bc(v2f[5:6]))
    return v0 + v1 + v2


def _factor_messages(fp, v2f):
    fb = fp + _expand_msgs(v2f)
    m = jnp.max(fb, axis=0, keepdims=True)
    e = jnp.exp(fb - m)
    s8 = jnp.sum(e, axis=0, keepdims=True)
    r = _rowi()
    t0 = jnp.sum(jnp.where((r >> 2) & 1 == 0, e, 0.0), axis=0, keepdims=True)
    t1 = jnp.sum(jnp.where((r >> 1) & 1 == 0, e, 0.0), axis=0, keepdims=True)
    t2 = jnp.sum(jnp.where(r & 1 == 0, e, 0.0), axis=0, keepdims=True)
    z = jnp.concatenate([t0, s8 - t0, t1, s8 - t1, t2, s8 - t2, s8, s8], axis=0)
    marg = m + jnp.log(z)
    msg = marg - v2f
    return _pair_normalize(msg) * _mk_mask67()


def _transform(g, f2vp, w_ref):
    u = g - f2vp
    r = _rowi()
    even = (r % 2) == 0
    valid = (r < 6).astype(jnp.float32)
    ue = jnp.where(even, u, jnp.roll(u, 1, axis=0))
    uo = jnp.where(even, jnp.roll(u, -1, axis=0), u)
    ce = jnp.where(even, w_ref[0, 0], w_ref[0, 1]) * valid
    co = jnp.where(even, w_ref[1, 0], w_ref[1, 1]) * valid
    v = u + ue * ce + uo * co
    return _pair_normalize(v) * _mk_mask67()


def _ka0_body(fp_ref, f2v_ref):
    fp = fp_ref[:, :]
    f2v_ref[:, :] = _factor_messages(fp, jnp.zeros_like(fp))


def _ka_body(fp_ref, g_ref, f2vp_ref, w_ref, f2v_ref):
    v2f = _transform(g_ref[:, :], f2vp_ref[:, :], w_ref)
    f2v_ref[:, :] = _factor_messages(fp_ref[:, :], v2f)


def _kc_body(p0_ref, p1_ref, p2_ref, p3_ref, p4_ref, p5_ref, p6_ref, p7_ref,
             n0_ref, n1_ref):
    vb0 = (p0_ref[:, :] + p2_ref[:, :]) + (p4_ref[:, :] + p6_ref[:, :])
    vb1 = (p1_ref[:, :] + p3_ref[:, :]) + (p5_ref[:, :] + p7_ref[:, :])
    m = jnp.maximum(vb0, vb1)
    l = m + jnp.log1p(jnp.exp(-jnp.abs(vb0 - vb1)))
    n0_ref[:, :] = vb0 - l
    n1_ref[:, :] = vb1 - l


def _kf_body(fp_ref, g_ref, f2vp_ref, w_ref, lim_ref, out_ref):
    fp = fp_ref[:, :]
    v2f = _transform(g_ref[:, :], f2vp_ref[:, :], w_ref)
    fb = fp + _expand_msgs(v2f)
    m = jnp.max(fb, axis=0, keepdims=True)
    e = jnp.exp(fb - m)
    l8 = m + jnp.log(jnp.sum(e, axis=0, keepdims=True))
    e8 = jnp.exp(fb - l8)
    t = e8 * (fp - fb + l8)
    pos = pl.program_id(0) * BF + lax.broadcasted_iota(jnp.int32, (8, BF), 1)
    part = jnp.sum(jnp.where(pos < lim_ref[0], t, 0.0))

    @pl.when(pl.program_id(0) == 0)
    def _():
        out_ref[0, 0] = 0.0

    out_ref[0, 0] += part


def _kcg_body(p0_ref, p1_ref, p2_ref, p3_ref, p4_ref, p5_ref, p6_ref, p7_ref,
              d0_ref, d1_ref, d2_ref, d3_ref, n0_ref, n1_ref, out_ref):
    vb0 = (p0_ref[:, :] + p2_ref[:, :]) + (p4_ref[:, :] + p6_ref[:, :])
    vb1 = (p1_ref[:, :] + p3_ref[:, :]) + (p5_ref[:, :] + p7_ref[:, :])
    m = jnp.maximum(vb0, vb1)
    l = m + jnp.log1p(jnp.exp(-jnp.abs(vb0 - vb1)))
    n0 = vb0 - l
    n1 = vb1 - l
    n0_ref[:, :] = n0
    n1_ref[:, :] = n1
    deg = (d0_ref[:, :] + d1_ref[:, :]) + (d2_ref[:, :] + d3_ref[:, :])
    inner = jnp.exp(n0) * n0 + jnp.exp(n1) * n1
    rows = pl.program_id(0) * 8 + lax.broadcasted_iota(jnp.int32, (8, 512), 0)
    pos = rows * 512 + lax.broadcasted_iota(jnp.int32, (8, 512), 1)
    part = jnp.sum(jnp.where(pos < N, (deg - 1.0) * inner, 0.0))

    @pl.when(pl.program_id(0) == 0)
    def _():
        out_ref[0, 0] = 0.0

    out_ref[0, 0] += part


_GF = FH // BF
_GN = NR // 8

_spec8 = pl.BlockSpec((8, BF), lambda i: (0, i))
_specn = pl.BlockSpec((8, 512), lambda i: (i, 0))
_spec11 = pl.BlockSpec((1, 1), lambda i: (0, 0), memory_space=pltpu.SMEM)
_specsm = pl.BlockSpec(memory_space=pltpu.SMEM)

_ka0 = pl.pallas_call(
    _ka0_body,
    grid=(_GF,),
    in_specs=[_spec8],
    out_specs=_spec8,
    out_shape=jax.ShapeDtypeStruct((8, FH), jnp.float32),
)

_ka = pl.pallas_call(
    _ka_body,
    grid=(_GF,),
    in_specs=[_spec8, _spec8, _spec8, _specsm],
    out_specs=_spec8,
    out_shape=jax.ShapeDtypeStruct((8, FH), jnp.float32),
)

_kc = pl.pallas_call(
    _kc_body,
    grid=(_GN,),
    in_specs=[_specn] * 8,
    out_specs=(_specn, _specn),
    out_shape=(jax.ShapeDtypeStruct((NR, 512), jnp.float32),
               jax.ShapeDtypeStruct((NR, 512), jnp.float32)),
)

_kf = pl.pallas_call(
    _kf_body,
    grid=(_GF,),
    in_specs=[_spec8, _spec8, _spec8, _specsm, _specsm],
    out_specs=_spec11,
    out_shape=jax.ShapeDtypeStruct((1, 1), jnp.float32),
)

_kcg = pl.pallas_call(
    _kcg_body,
    grid=(_GN,),
    in_specs=[_specn] * 12,
    out_specs=(_specn, _specn, _spec11),
    out_shape=(jax.ShapeDtypeStruct((NR, 512), jnp.float32),
               jax.ShapeDtypeStruct((NR, 512), jnp.float32),
               jax.ShapeDtypeStruct((1, 1), jnp.float32)),
)


_sc_mesh = plsc.VectorSubcoreMesh(core_axis_name="c", subcore_axis_name="s")


def _zero_fill(buf, n):
    def body(i, carry):
        buf[pl.ds(i * 16, 16)] = jnp.zeros((16,), jnp.float32)
        return carry
    lax.fori_loop(0, n // 16, body, 0, unroll=4)


def _flatten_row(src2d, r, dst1d, n):
    def body(i, carry):
        dst1d[pl.ds(i * 16, 16)] = src2d[r, pl.ds(i * 16, 16)]
        return carry
    lax.fori_loop(0, n // 16, body, 0, unroll=4)


@functools.partial(
    pl.kernel, mesh=_sc_mesh,
    out_type=tuple(jax.ShapeDtypeStruct((N_PAD,), jnp.float32)
                   for _ in range(4)),
    scratch_types=[
        pltpu.VMEM((8, CHE), jnp.float32),
        pltpu.VMEM((CHE,), jnp.int32),
        pltpu.VMEM((CHE,), jnp.float32),
        pltpu.VMEM((CHE,), jnp.float32),
        pltpu.VMEM((CHN,), jnp.float32),
        pltpu.VMEM_SHARED((N_PAD,), jnp.float32),
        pltpu.VMEM_SHARED((N_PAD,), jnp.float32),
    ],
)
def _scatter(f2v_hbm, idx0_hbm, idx1_hbm, idx2_hbm,
             o0_hbm, o1_hbm, o2_hbm, o3_hbm,
             fbuf, idx_v, v0, v1, zbuf, acc0, acc1):
    c = lax.axis_index("c")
    s = lax.axis_index("s")
    w = c * NS + s
    _zero_fill(zbuf, CHN)
    pltpu.sync_copy(zbuf, acc0.at[pl.ds(s * CHN, CHN)])
    pltpu.sync_copy(zbuf, acc1.at[pl.ds(s * CHN, CHN)])
    plsc.subcore_barrier()
    base = w * CHE
    pltpu.sync_copy(f2v_hbm.at[:, pl.ds(base, CHE)], fbuf)
    for d, idx_hbm in enumerate((idx0_hbm, idx1_hbm, idx2_hbm)):
        pltpu.sync_copy(idx_hbm.at[pl.ds(base, CHE)], idx_v)
        _flatten_row(fbuf, 2 * d, v0, CHE)
        _flatten_row(fbuf, 2 * d + 1, v1, CHE)
        pltpu.sync_copy(v0, acc0.at[idx_v], add=True)
        pltpu.sync_copy(v1, acc1.at[idx_v], add=True)
    plsc.subcore_barrier()

    @pl.when(c == 0)
    def _():
        pltpu.sync_copy(acc0.at[pl.ds(s * CHN, CHN)], o0_hbm.at[pl.ds(s * CHN, CHN)])
        pltpu.sync_copy(acc1.at[pl.ds(s * CHN, CHN)], o1_hbm.at[pl.ds(s * CHN, CHN)])

    @pl.when(c == 1)
    def _():
        pltpu.sync_copy(acc0.at[pl.ds(s * CHN, CHN)], o2_hbm.at[pl.ds(s * CHN, CHN)])
        pltpu.sync_copy(acc1.at[pl.ds(s * CHN, CHN)], o3_hbm.at[pl.ds(s * CHN, CHN)])


@functools.partial(
    pl.kernel, mesh=_sc_mesh,
    out_type=jax.ShapeDtypeStruct((8, FH), jnp.float32),
    scratch_types=[
        pltpu.VMEM((8, CHE), jnp.float32),
        pltpu.VMEM((CHE,), jnp.int32),
        pltpu.VMEM((CHE,), jnp.float32),
        pltpu.VMEM((CHN,), jnp.float32),
        pltpu.VMEM_SHARED((N_PAD,), jnp.float32),
        pltpu.VMEM_SHARED((N_PAD,), jnp.float32),
        pltpu.SemaphoreType.DMA,
    ],
)
def _gather(n0_hbm, n1_hbm, idx0_hbm, idx1_hbm, idx2_hbm, g_hbm,
            gbuf, idx_v, g1, tmp, sh0, sh1, sem):
    c = lax.axis_index("c")
    s = lax.axis_index("s")
    w = c * NS + s
    pltpu.sync_copy(n0_hbm.at[pl.ds(s * CHN, CHN)], tmp)
    pltpu.sync_copy(tmp, sh0.at[pl.ds(s * CHN, CHN)])
    pltpu.sync_copy(n1_hbm.at[pl.ds(s * CHN, CHN)], tmp)
    pltpu.sync_copy(tmp, sh1.at[pl.ds(s * CHN, CHN)])
    def zrow(i, carry):
        z = jnp.zeros((16,), jnp.float32)
        gbuf[6, pl.ds(i * 16, 16)] = z
        gbuf[7, pl.ds(i * 16, 16)] = z
        return carry
    lax.fori_loop(0, CHE // 16, zrow, 0, unroll=4)
    plsc.subcore_barrier()
    base = w * CHE
    for d, idx_hbm in enumerate((idx0_hbm, idx1_hbm, idx2_hbm)):
        pltpu.sync_copy(idx_hbm.at[pl.ds(base, CHE)], idx_v)
        pltpu.async_copy(sh0.at[idx_v], g1, sem).wait()

        def cp0(i, carry):
            gbuf[2 * d, pl.ds(i * 16, 16)] = g1[pl.ds(i * 16, 16)]
            return carry
        lax.fori_loop(0, CHE // 16, cp0, 0, unroll=4)
        pltpu.async_copy(sh1.at[idx_v], g1, sem).wait()

        def cp1(i, carry):
            gbuf[2 * d + 1, pl.ds(i * 16, 16)] = g1[pl.ds(i * 16, 16)]
            return carry
        lax.fori_loop(0, CHE // 16, cp1, 0, unroll=4)
    pltpu.sync_copy(gbuf, g_hbm.at[:, pl.ds(base, CHE)])


@functools.partial(
    pl.kernel, mesh=_sc_mesh,
    out_type=tuple(jax.ShapeDtypeStruct((N_PAD,), jnp.float32)
                   for _ in range(6)),
    scratch_types=[
        pltpu.VMEM((8, CHE), jnp.float32),
        pltpu.VMEM((CHE,), jnp.int32),
        pltpu.VMEM((CHE,), jnp.float32),
        pltpu.VMEM((CHE,), jnp.float32),
        pltpu.VMEM((CHE,), jnp.float32),
        pltpu.VMEM((CHN,), jnp.float32),
        pltpu.VMEM_SHARED((N_PAD,), jnp.float32),
        pltpu.VMEM_SHARED((N_PAD,), jnp.float32),
        pltpu.VMEM_SHARED((N_PAD,), jnp.float32),
    ],
)
def _scatter_deg(f2v_hbm, idx0_hbm, idx1_hbm, idx2_hbm, ones_hbm,
                 o0_hbm, o1_hbm, o2_hbm, o3_hbm, d0_hbm, d1_hbm,
                 fbuf, idx_v, v0, v1, ones_v, zbuf, acc0, acc1, acc2):
    c = lax.axis_index("c")
    s = lax.axis_index("s")
    w = c * NS + s
    _zero_fill(zbuf, CHN)
    pltpu.sync_copy(zbuf, acc0.at[pl.ds(s * CHN, CHN)])
    pltpu.sync_copy(zbuf, acc1.at[pl.ds(s * CHN, CHN)])
    pltpu.sync_copy(zbuf, acc2.at[pl.ds(s * CHN, CHN)])
    plsc.subcore_barrier()
    base = w * CHE
    pltpu.sync_copy(f2v_hbm.at[:, pl.ds(base, CHE)], fbuf)
    pltpu.sync_copy(ones_hbm.at[pl.ds(base, CHE)], ones_v)
    for d, idx_hbm in enumerate((idx0_hbm, idx1_hbm, idx2_hbm)):
        pltpu.sync_copy(idx_hbm.at[pl.ds(base, CHE)], idx_v)
        _flatten_row(fbuf, 2 * d, v0, CHE)
        _flatten_row(fbuf, 2 * d + 1, v1, CHE)
        pltpu.sync_copy(v0, acc0.at[idx_v], add=True)
        pltpu.sync_copy(v1, acc1.at[idx_v], add=True)
        pltpu.sync_copy(ones_v, acc2.at[idx_v], add=True)
    plsc.subcore_barrier()

    @pl.when(c == 0)
    def _():
        pltpu.sync_copy(acc0.at[pl.ds(s * CHN, CHN)], o0_hbm.at[pl.ds(s * CHN, CHN)])
        pltpu.sync_copy(acc1.at[pl.ds(s * CHN, CHN)], o1_hbm.at[pl.ds(s * CHN, CHN)])
        pltpu.sync_copy(acc2.at[pl.ds(s * CHN, CHN)], d0_hbm.at[pl.ds(s * CHN, CHN)])

    @pl.when(c == 1)
    def _():
        pltpu.sync_copy(acc0.at[pl.ds(s * CHN, CHN)], o2_hbm.at[pl.ds(s * CHN, CHN)])
        pltpu.sync_copy(acc1.at[pl.ds(s * CHN, CHN)], o3_hbm.at[pl.ds(s * CHN, CHN)])
        pltpu.sync_copy(acc2.at[pl.ds(s * CHN, CHN)], d1_hbm.at[pl.ds(s * CHN, CHN)])


def kernel(factor_potentials, edge_var_indices, layer_weights):
    fp_t = jnp.pad(factor_potentials.reshape(F, 8).T, ((0, 0), (0, F_PAD - F)))
    idx3 = edge_var_indices.reshape(F, 3).T
    pad = F_PAD - F
    idxf = [jnp.pad(idx3[k], (0, pad)) for k in range(3)]
    ones_mask = (jnp.arange(F_PAD) < F).astype(jnp.float32)

    fph = (fp_t[:, :FH], fp_t[:, FH:])
    idxh = tuple(tuple(ix[h * FH:(h + 1) * FH] for ix in idxf)
                 for h in range(2))
    onesh = (ones_mask[:FH], ones_mask[FH:])
    limh = (jnp.full((1,), FH, jnp.int32), jnp.full((1,), F - FH, jnp.int32))

    degp = None
    s_n = None
    g = (None, None)
    f2v = tuple(_ka0(fph[h]) for h in range(2))
    for it in range(ITERS):
        if it > 0:
            f2v = tuple(_ka(fph[h], g[h], f2v[h], layer_weights[it - 1])
                        for h in range(2))
        if it == 0:
            outs = [_scatter_deg(f2v[h], *idxh[h], onesh[h]) for h in range(2)]
            vbp = outs[0][:4] + outs[1][:4]
            degp = [d.reshape(NR, 512) for d in outs[0][4:] + outs[1][4:]]
        else:
            vbp = (_scatter(f2v[0], *idxh[0]) + _scatter(f2v[1], *idxh[1]))
        vbp2 = [p.reshape(NR, 512) for p in vbp]
        if it == ITERS - 1:
            n0, n1, s_n = _kcg(*vbp2, *degp)
        else:
            n0, n1 = _kc(*vbp2)
        n0f = n0.reshape(N_PAD)
        n1f = n1.reshape(N_PAD)
        g = tuple(_gather(n0f, n1f, *idxh[h]) for h in range(2))
    wlast = layer_weights[ITERS - 1]
    s_f0 = _kf(fph[0], g[0], f2v[0], wlast, limh[0])
    s_f1 = _kf(fph[1], g[1], f2v[1], wlast, limh[1])
    return (s_f0[0, 0] + s_f1[0, 0] + s_n[0, 0]).astype(jnp.float32)
